# baseline (speedup 1.0000x reference)
import functools

import jax
import jax.numpy as jnp
from jax import lax
from jax.experimental import pallas as pl
from jax.experimental.pallas import tpu as pltpu
from jax.experimental.pallas import tpu_sc as plsc

L = 16
NC, NS = 2, 16
NW = NC * NS
BATCH = 256
H, W = 513, 257
HS = 16
NSTRIPE = H // HS
WV = W // L
NSH = 257
NSP = 272
RPW = BATCH // NW
EPS = 1e-5


def _row_vecs(x_buf, w_buf, idx_buf, acc, hh, tail_mask):
    for v in range(WV):
        o = v * L
        wv = w_buf[0, hh, pl.ds(o, L)]
        iv = idx_buf[0, hh, pl.ds(o, L)]
        for r in range(RPW):
            xv = x_buf[r, hh, pl.ds(o, L)]
            yv = xv * xv * wv
            plsc.addupdate_scatter(acc, [iv + (r * NSP)], yv)
    o = W - L
    wv = w_buf[0, hh, pl.ds(o, L)]
    iv = idx_buf[0, hh, pl.ds(o, L)]
    for r in range(RPW):
        xv = x_buf[r, hh, pl.ds(o, L)]
        yv = xv * xv * wv
        plsc.addupdate_scatter(acc, [iv + (r * NSP)], yv, mask=tail_mask)


def _body(x_hbm, w_hbm, idx_hbm, cnt_hbm, out_hbm,
          x_buf, w_buf, idx_buf, xr_buf, wr_buf, ir_buf,
          acc, cnt_buf, rec, out_buf):
    wid = lax.axis_index("s") * NC + lax.axis_index("c")
    row0 = wid * RPW
    tail_mask = lax.iota(jnp.int32, L) == (L - 1)

    zeros = jnp.zeros((L,), jnp.float32)

    def zbody(i, c):
        acc[pl.ds(i * L, L)] = zeros
        return c

    lax.fori_loop(0, (RPW * NSP) // L, zbody, 0)

    def sbody(s, carry):
        h0 = pl.multiple_of(s * HS, HS)
        pltpu.sync_copy(x_hbm.at[pl.ds(row0, RPW), pl.ds(h0, HS)], x_buf)
        pltpu.sync_copy(w_hbm.at[pl.ds(0, 1), pl.ds(h0, HS)], w_buf)
        pltpu.sync_copy(idx_hbm.at[pl.ds(0, 1), pl.ds(h0, HS)], idx_buf)

        def hbody(hh, c):
            _row_vecs(x_buf, w_buf, idx_buf, acc, hh, tail_mask)
            return c

        lax.fori_loop(0, HS, hbody, 0)
        return carry

    lax.fori_loop(0, NSTRIPE, sbody, 0)

    pltpu.sync_copy(x_hbm.at[pl.ds(row0, RPW), pl.ds(H - 1, 1)], xr_buf)
    pltpu.sync_copy(w_hbm.at[pl.ds(0, 1), pl.ds(H - 1, 1)], wr_buf)
    pltpu.sync_copy(idx_hbm.at[pl.ds(0, 1), pl.ds(H - 1, 1)], ir_buf)
    _row_vecs(xr_buf, wr_buf, ir_buf, acc, 0, tail_mask)

    pltpu.sync_copy(cnt_hbm, cnt_buf)
    for v in range(NSP // L):
        o = v * L
        rec[pl.ds(o, L)] = 1.0 / (cnt_buf[pl.ds(o, L)] + EPS)
    for r in range(RPW):
        for v in range(NSP // L):
            o = v * L
            out_buf[r, pl.ds(o, L)] = acc[pl.ds(r * NSP + o, L)] * rec[pl.ds(o, L)]
    pltpu.sync_copy(out_buf, out_hbm.at[pl.ds(row0, RPW)])


@jax.jit
def _sc_spectrum(x3, w2, idx2, cnt):
    mesh = plsc.VectorSubcoreMesh(core_axis_name="c", subcore_axis_name="s")
    f = pl.kernel(
        _body,
        mesh=mesh,
        compiler_params=pltpu.CompilerParams(
            needs_layout_passes=False, use_tc_tiling_on_sc=False),
        out_type=jax.ShapeDtypeStruct((BATCH, NSP), jnp.float32),
        scratch_types=[
            pltpu.VMEM((RPW, HS, W), jnp.float32),
            pltpu.VMEM((1, HS, W), jnp.float32),
            pltpu.VMEM((1, HS, W), jnp.int32),
            pltpu.VMEM((RPW, 1, W), jnp.float32),
            pltpu.VMEM((1, 1, W), jnp.float32),
            pltpu.VMEM((1, 1, W), jnp.int32),
            pltpu.VMEM((RPW * NSP,), jnp.float32),
            pltpu.VMEM((NSP,), jnp.float32),
            pltpu.VMEM((NSP,), jnp.float32),
            pltpu.VMEM((RPW, NSP), jnp.float32),
        ],
    )
    return f(x3, w2, idx2, cnt)


def kernel(x, shells_weight, shell_index, shells_count):
    b, c, h, w_ = x.shape
    x3 = x.reshape(b, h, w_)
    w2 = shells_weight.reshape(1, h, w_)
    idx2 = shell_index.reshape(1, h, w_)
    cnt = jnp.concatenate(
        [shells_count, jnp.ones((NSP - NSH,), jnp.float32)])
    out = _sc_spectrum(x3, w2, idx2, cnt)
    return out[:, :NSH].reshape(b, c, NSH)
